# baseline (speedup 1.0000x reference)
import jax
import jax.numpy as jnp
from jax.experimental import pallas as pl
from jax.experimental.pallas import tpu as pltpu

_ALPHA = 0.2
_NEG = -9e15
_NQ = 4


def _gat_body(x_ref, adj_hbm, w_ref, b_ref, a_ref, out_ref, adj_ref, sems):
    n = x_ref.shape[1]
    q = n // _NQ
    c_out = w_ref.shape[0]
    a1 = a_ref[:, :c_out]
    a2 = a_ref[:, c_out:]

    cps = []
    for k in range(_NQ):
        cp = pltpu.make_async_copy(
            adj_hbm.at[0, pl.ds(k * q, q), :],
            adj_ref.at[pl.ds(k * q, q), :],
            sems.at[k],
        )
        cp.start()
        cps.append(cp)

    nf = jax.lax.dot_general(
        x_ref[0], w_ref[...], (((1,), (1,)), ((), ())),
        preferred_element_type=jnp.float32,
    ) + b_ref[...]
    s2 = jax.lax.dot_general(
        a2, nf, (((1,), (1,)), ((), ())),
        preferred_element_type=jnp.float32,
    )
    s1 = jax.lax.dot_general(
        nf, a1, (((1,), (1,)), ((), ())),
        preferred_element_type=jnp.float32,
    )

    for k in range(_NQ):
        cps[k].wait()
        logits = s1[k * q:(k + 1) * q, :] + s2
        leaky = jnp.maximum(logits, _ALPHA * logits)
        masked = jnp.where(adj_ref[k * q:(k + 1) * q, :] != 0, leaky, _NEG)
        m = jnp.max(masked, axis=1, keepdims=True)
        e = jnp.exp(masked - m)
        denom = jnp.sum(e, axis=1, keepdims=True)
        acc = jax.lax.dot_general(
            e, nf, (((1,), (0,)), ((), ())),
            preferred_element_type=jnp.float32,
        )
        out_ref[0, k * q:(k + 1) * q, :] = acc / denom


def kernel(node_feats, adj_matrix, W, b, a):
    if node_feats.ndim == 2:
        node_feats = node_feats[None]
    B, N, C_IN = node_feats.shape
    C_OUT = W.shape[0]
    out = pl.pallas_call(
        _gat_body,
        in_specs=[
            pl.BlockSpec((B, N, C_IN), lambda: (0, 0, 0)),
            pl.BlockSpec(memory_space=pltpu.MemorySpace.HBM),
            pl.BlockSpec((C_OUT, C_IN), lambda: (0, 0)),
            pl.BlockSpec((C_OUT,), lambda: (0,)),
            pl.BlockSpec((1, 2 * C_OUT), lambda: (0, 0)),
        ],
        out_specs=pl.BlockSpec((B, N, C_OUT), lambda: (0, 0, 0)),
        out_shape=jax.ShapeDtypeStruct((B, N, C_OUT), jnp.float32),
        scratch_shapes=[
            pltpu.VMEM((N, N), adj_matrix.dtype),
            pltpu.SemaphoreType.DMA((_NQ,)),
        ],
    )(node_feats, adj_matrix, W, b, a)
    return out


# speedup vs baseline: 1.0971x; 1.0971x over previous
import jax
import jax.numpy as jnp
from jax.experimental import pallas as pl
from jax.experimental.pallas import tpu as pltpu

_ALPHA = 0.2
_NEG = -9e15
_BLK = 512


def _gat_body(x_ref, adj_ref, w_ref, b_ref, a_ref, out_ref, nf_ref, s2_ref):
    i = pl.program_id(0)
    c_out = w_ref.shape[0]
    a1 = a_ref[:, :c_out]
    a2 = a_ref[:, c_out:]

    @pl.when(i == 0)
    def _():
        nf = jax.lax.dot_general(
            x_ref[0], w_ref[...], (((1,), (1,)), ((), ())),
            preferred_element_type=jnp.float32,
        ) + b_ref[...]
        nf_ref[...] = nf
        s2_ref[...] = jax.lax.dot_general(
            a2, nf, (((1,), (1,)), ((), ())),
            preferred_element_type=jnp.float32,
        )

    nf = nf_ref[...]
    nfb = nf_ref[pl.ds(i * _BLK, _BLK), :]
    s1 = jax.lax.dot_general(
        nfb, a1, (((1,), (1,)), ((), ())),
        preferred_element_type=jnp.float32,
    )
    logits = s1 + s2_ref[...]
    leaky = jnp.maximum(logits, _ALPHA * logits)
    masked = jnp.where(adj_ref[0] != 0, leaky, _NEG)
    m = jnp.max(masked, axis=1, keepdims=True)
    e = jnp.exp(masked - m)
    denom = jnp.sum(e, axis=1, keepdims=True)
    acc = jax.lax.dot_general(
        e, nf, (((1,), (0,)), ((), ())),
        preferred_element_type=jnp.float32,
    )
    out_ref[0] = acc / denom


def kernel(node_feats, adj_matrix, W, b, a):
    if node_feats.ndim == 2:
        node_feats = node_feats[None]
    B, N, C_IN = node_feats.shape
    C_OUT = W.shape[0]
    nblk = N // _BLK
    out = pl.pallas_call(
        _gat_body,
        grid=(nblk,),
        in_specs=[
            pl.BlockSpec((1, N, C_IN), lambda i: (0, 0, 0)),
            pl.BlockSpec((1, _BLK, N), lambda i: (0, i, 0)),
            pl.BlockSpec((C_OUT, C_IN), lambda i: (0, 0)),
            pl.BlockSpec((C_OUT,), lambda i: (0,)),
            pl.BlockSpec((1, 2 * C_OUT), lambda i: (0, 0)),
        ],
        out_specs=pl.BlockSpec((1, _BLK, C_OUT), lambda i: (0, i, 0)),
        out_shape=jax.ShapeDtypeStruct((B, N, C_OUT), jnp.float32),
        scratch_shapes=[
            pltpu.VMEM((N, C_OUT), jnp.float32),
            pltpu.VMEM((1, N), jnp.float32),
        ],
    )(node_feats, adj_matrix, W, b, a)
    return out
